# baseline (speedup 1.0000x reference)
import jax
import jax.numpy as jnp
from jax import lax
from jax.experimental import pallas as pl
from jax.experimental.pallas import tpu as pltpu
from jax.experimental.pallas import tpu_sc as plsc

_INFO = plsc.get_sparse_core_info()
_NC = _INFO.num_cores
_NS = _INFO.num_subcores
_NW = _NC * _NS


def _copy_rows(src, dst, buf, off, s):
    pltpu.sync_copy(src.at[pl.ds(off, s), :], dst.at[pl.ds(off, s), :])


def _sc_copy_body(src, out, buf):
    wid = lax.axis_index("s") * _NC + lax.axis_index("c")
    n = src.shape[0]
    base, rem = divmod(n // 8, _NW)
    rows_big = (base + 1) * 8
    rows_small = base * 8
    if rem:
        @pl.when(wid < rem)
        def _():
            _copy_rows(src, out, buf, wid * rows_big, rows_big)

        @pl.when(wid >= rem)
        def _():
            off = rem * rows_big + (wid - rem) * rows_small
            _copy_rows(src, out, buf, off, rows_small)
    else:
        _copy_rows(src, out, buf, wid * rows_small, rows_small)


def kernel(variable_features, constraint_features, edge_indices, reversed_edge_indices):
    n, d = variable_features.shape
    ch = (480_000 // (d * 4)) // 8 * 8
    mesh = plsc.VectorSubcoreMesh(core_axis_name="c", subcore_axis_name="s")
    out = pl.kernel(
        _sc_copy_body,
        out_type=jax.ShapeDtypeStruct((n, d), variable_features.dtype),
        mesh=mesh,
        scratch_types=[pltpu.VMEM((ch, d), jnp.float32)],
        compiler_params=pltpu.CompilerParams(use_tc_tiling_on_sc=True,
                                             skip_device_barrier=True),
    )(variable_features)
    return out


# speedup vs baseline: 21.5722x; 21.5722x over previous
import jax
import jax.numpy as jnp
from jax import lax
from jax.experimental import pallas as pl
from jax.experimental.pallas import tpu as pltpu
from jax.experimental.pallas import tpu_sc as plsc

_INFO = plsc.get_sparse_core_info()
_NC = _INFO.num_cores
_NS = _INFO.num_subcores
_NW = _NC * _NS


def _copy_rows(src, dst, buf, off, s):
    ch = buf.shape[0]
    done = 0
    while done < s:
        c = min(ch, s - done)
        pltpu.sync_copy(src.at[pl.ds(off + done, c), :], buf.at[pl.ds(0, c), :])
        pltpu.sync_copy(buf.at[pl.ds(0, c), :], dst.at[pl.ds(off + done, c), :])
        done += c


def _sc_copy_body(src, out, buf):
    wid = lax.axis_index("s") * _NC + lax.axis_index("c")
    n = src.shape[0]
    base, rem = divmod(n // 8, _NW)
    rows_big = (base + 1) * 8
    rows_small = base * 8
    if rem:
        @pl.when(wid < rem)
        def _():
            _copy_rows(src, out, buf, wid * rows_big, rows_big)

        @pl.when(wid >= rem)
        def _():
            off = rem * rows_big + (wid - rem) * rows_small
            _copy_rows(src, out, buf, off, rows_small)
    else:
        _copy_rows(src, out, buf, wid * rows_small, rows_small)


def kernel(variable_features, constraint_features, edge_indices, reversed_edge_indices):
    n, d = variable_features.shape
    ch = (520_000 // (d * 4)) // 8 * 8
    mesh = plsc.VectorSubcoreMesh(core_axis_name="c", subcore_axis_name="s")
    out = pl.kernel(
        _sc_copy_body,
        out_type=jax.ShapeDtypeStruct((n, d), variable_features.dtype),
        mesh=mesh,
        scratch_types=[pltpu.VMEM((ch, d), jnp.float32)],
        compiler_params=pltpu.CompilerParams(use_tc_tiling_on_sc=True),
    )(variable_features)
    return out


# speedup vs baseline: 21.8516x; 1.0130x over previous
import jax
import jax.numpy as jnp
from jax import lax
from jax.experimental import pallas as pl
from jax.experimental.pallas import tpu as pltpu
from jax.experimental.pallas import tpu_sc as plsc

_INFO = plsc.get_sparse_core_info()
_NC = _INFO.num_cores
_NS = _INFO.num_subcores
_NW = _NC * _NS


def _sc_copy_body(src_hbm, out_hbm, buf):
    wid = lax.axis_index("s") * _NC + lax.axis_index("c")
    n = src_hbm.shape[0] // _NW
    chunk = buf.shape[0]
    base = wid * n
    for i in range(n // chunk):
        off = base + i * chunk
        pltpu.sync_copy(src_hbm.at[pl.ds(off, chunk)], buf)
        pltpu.sync_copy(buf, out_hbm.at[pl.ds(off, chunk)])


def kernel(variable_features, constraint_features, edge_indices, reversed_edge_indices):
    n_var, d = variable_features.shape
    flat = variable_features.reshape(-1)
    per_worker = flat.shape[0] // _NW
    chunk = per_worker
    while chunk * 4 > 400_000:
        chunk //= 2
    mesh = plsc.VectorSubcoreMesh(core_axis_name="c", subcore_axis_name="s")
    out = pl.kernel(
        _sc_copy_body,
        out_type=jax.ShapeDtypeStruct(flat.shape, flat.dtype),
        mesh=mesh,
        scratch_types=[pltpu.VMEM((chunk,), jnp.float32)],
    )(flat)
    return out.reshape(n_var, d)
